# baseline (speedup 1.0000x reference)
import functools

import jax
import jax.numpy as jnp
from jax import lax
from jax.experimental import pallas as pl
from jax.experimental.pallas import tpu as pltpu
from jax.experimental.pallas import tpu_sc as plsc

VOCAB = 100000
EMB = 128
HID = 1024
B, L = 4, 2048
PAD = 8
PADL = L + 2 * PAD
NW = 32
CH = (B * L) // NW
TL = 512


def _sc_gather(ids_flat, table):
    mesh = plsc.VectorSubcoreMesh(core_axis_name="c", subcore_axis_name="s")

    @functools.partial(
        pl.kernel,
        mesh=mesh,
        out_type=jax.ShapeDtypeStruct((B * PADL, EMB), jnp.float32),
        scratch_types=[
            pltpu.VMEM((CH,), jnp.int32),
            pltpu.VMEM((CH, EMB), jnp.float32),
            pltpu.VMEM((PAD, EMB), jnp.float32),
            pltpu.SemaphoreType.DMA,
        ],
    )
    def gather_kernel(idx_hbm, table_hbm, out_hbm, idx_v, rows_v, zero_v, sem):
        cid = lax.axis_index("c")
        sid = lax.axis_index("s")
        wid = cid * 16 + sid
        fb = wid * CH
        b = fb // L
        out_row = b * PADL + PAD + (fb - b * L)
        pltpu.sync_copy(idx_hbm.at[pl.ds(fb, CH)], idx_v)
        pltpu.async_copy(table_hbm.at[idx_v], rows_v, sem).wait()
        pltpu.sync_copy(rows_v, out_hbm.at[pl.ds(out_row, CH)])
        z = jnp.zeros((16,), jnp.float32)
        for i in range(PAD):
            for j in range(EMB // 16):
                zero_v[i, pl.ds(j * 16, 16)] = z
        zb = wid // 2
        zrow = zb * PADL + (wid % 2) * (PAD + L)

        @pl.when(wid < 2 * B)
        def _():
            pltpu.sync_copy(zero_v, out_hbm.at[pl.ds(zrow, PAD)])

    return gather_kernel(ids_flat, table)


def _tc_body(epad_ref, w_ref, b_ref, pos_ref, type_ref, lnw_ref, lnb_ref, out_ref):
    l = pl.program_id(1)
    base = PAD + l * TL
    ec = epad_ref[0, pl.ds(base, TL), :].astype(jnp.bfloat16)
    el = epad_ref[0, pl.ds(base + 1, TL), :].astype(jnp.bfloat16)
    er = epad_ref[0, pl.ds(base - 1, TL), :].astype(jnp.bfloat16)
    w = w_ref[...]
    h = jnp.dot(el, w[0:EMB, :], preferred_element_type=jnp.float32)
    h += jnp.dot(ec, w[EMB:2 * EMB, :], preferred_element_type=jnp.float32)
    h += jnp.dot(er, w[2 * EMB:3 * EMB, :], preferred_element_type=jnp.float32)
    h += b_ref[...] + pos_ref[...] + type_ref[...]
    out_ref[0] = h * lnw_ref[...] + lnb_ref[...]


def kernel(input_ids, word_embeddings, dense_W, dense_b, pos_emb, type_emb,
           ln_weight, ln_bias):
    ids_flat = input_ids.reshape(-1).astype(jnp.int32)
    epad = _sc_gather(ids_flat, word_embeddings)
    epad = epad.reshape(B, PADL, EMB)

    grid = (B, L // TL)
    out = pl.pallas_call(
        _tc_body,
        grid=grid,
        in_specs=[
            pl.BlockSpec((1, PADL, EMB), lambda b, l: (b, 0, 0)),
            pl.BlockSpec((3 * EMB, HID), lambda b, l: (0, 0)),
            pl.BlockSpec((1, HID), lambda b, l: (0, 0)),
            pl.BlockSpec((TL, HID), lambda b, l: (l, 0)),
            pl.BlockSpec((1, HID), lambda b, l: (0, 0)),
            pl.BlockSpec((1, HID), lambda b, l: (0, 0)),
            pl.BlockSpec((1, HID), lambda b, l: (0, 0)),
        ],
        out_specs=pl.BlockSpec((1, TL, HID), lambda b, l: (b, l, 0)),
        out_shape=jax.ShapeDtypeStruct((B, L, HID), jnp.float32),
    )(
        epad,
        dense_W.astype(jnp.bfloat16),
        dense_b.reshape(1, HID),
        pos_emb,
        type_emb[0].reshape(1, HID),
        ln_weight.reshape(1, HID),
        ln_bias.reshape(1, HID),
    )
    return out


# speedup vs baseline: 1.1656x; 1.1656x over previous
import functools

import jax
import jax.numpy as jnp
from jax import lax
from jax.experimental import pallas as pl
from jax.experimental.pallas import tpu as pltpu
from jax.experimental.pallas import tpu_sc as plsc

VOCAB = 100000
EMB = 128
HID = 1024
B, L = 4, 2048
PAD = 8
PADL = L + 2 * PAD
NW = 32
CH = (B * L) // NW
TL = L


def _sc_gather(ids_flat, table):
    mesh = plsc.VectorSubcoreMesh(core_axis_name="c", subcore_axis_name="s")

    @functools.partial(
        pl.kernel,
        mesh=mesh,
        out_type=jax.ShapeDtypeStruct((B * PADL, EMB), jnp.float32),
        scratch_types=[
            pltpu.VMEM((CH,), jnp.int32),
            pltpu.VMEM((CH, EMB), jnp.float32),
            pltpu.VMEM((PAD, EMB), jnp.float32),
            pltpu.SemaphoreType.DMA,
        ],
    )
    def gather_kernel(idx_hbm, table_hbm, out_hbm, idx_v, rows_v, zero_v, sem):
        cid = lax.axis_index("c")
        sid = lax.axis_index("s")
        wid = cid * 16 + sid
        fb = wid * CH
        b = fb // L
        out_row = b * PADL + PAD + (fb - b * L)
        pltpu.sync_copy(idx_hbm.at[pl.ds(fb, CH)], idx_v)
        pltpu.async_copy(table_hbm.at[idx_v], rows_v, sem).wait()
        pltpu.sync_copy(rows_v, out_hbm.at[pl.ds(out_row, CH)])
        z = jnp.zeros((16,), jnp.float32)
        for i in range(PAD):
            for j in range(EMB // 16):
                zero_v[i, pl.ds(j * 16, 16)] = z
        zb = wid // 2
        zrow = zb * PADL + (wid % 2) * (PAD + L)

        @pl.when(wid < 2 * B)
        def _():
            pltpu.sync_copy(zero_v, out_hbm.at[pl.ds(zrow, PAD)])

    return gather_kernel(ids_flat, table)


def _tc_body(epad_ref, w_ref, b_ref, pos_ref, type_ref, lnw_ref, lnb_ref, out_ref):
    ec = epad_ref[0, pl.ds(PAD, TL), :].astype(jnp.bfloat16)
    el = epad_ref[0, pl.ds(PAD + 1, TL), :].astype(jnp.bfloat16)
    er = epad_ref[0, pl.ds(PAD - 1, TL), :].astype(jnp.bfloat16)
    w = w_ref[...]
    h = jnp.dot(el, w[0:EMB, :], preferred_element_type=jnp.float32)
    h += jnp.dot(ec, w[EMB:2 * EMB, :], preferred_element_type=jnp.float32)
    h += jnp.dot(er, w[2 * EMB:3 * EMB, :], preferred_element_type=jnp.float32)
    h += b_ref[...] + pos_ref[...] + type_ref[...]
    out_ref[0] = h * lnw_ref[...] + lnb_ref[...]


def kernel(input_ids, word_embeddings, dense_W, dense_b, pos_emb, type_emb,
           ln_weight, ln_bias):
    ids_flat = input_ids.reshape(-1).astype(jnp.int32)
    epad = _sc_gather(ids_flat, word_embeddings)
    epad = epad.reshape(B, PADL, EMB)

    grid = (B,)
    out = pl.pallas_call(
        _tc_body,
        grid=grid,
        in_specs=[
            pl.BlockSpec((1, PADL, EMB), lambda b: (b, 0, 0)),
            pl.BlockSpec((3 * EMB, HID), lambda b: (0, 0)),
            pl.BlockSpec((1, HID), lambda b: (0, 0)),
            pl.BlockSpec((TL, HID), lambda b: (0, 0)),
            pl.BlockSpec((1, HID), lambda b: (0, 0)),
            pl.BlockSpec((1, HID), lambda b: (0, 0)),
            pl.BlockSpec((1, HID), lambda b: (0, 0)),
        ],
        out_specs=pl.BlockSpec((1, TL, HID), lambda b: (b, 0, 0)),
        out_shape=jax.ShapeDtypeStruct((B, L, HID), jnp.float32),
    )(
        epad,
        dense_W.astype(jnp.bfloat16),
        dense_b.reshape(1, HID),
        pos_emb,
        type_emb[0].reshape(1, HID),
        ln_weight.reshape(1, HID),
        ln_bias.reshape(1, HID),
    )
    return out
